# Initial kernel scaffold; baseline (speedup 1.0000x reference)
import jax
import jax.numpy as jnp
from jax.experimental import pallas as pl


def kernel(x, embedding_table):
    raise NotImplementedError("write your pallas kernel here")


# speedup vs baseline: 5.2969x; 5.2969x over previous
import jax
import jax.numpy as jnp
from jax.experimental import pallas as pl
from jax.experimental.pallas import tpu as pltpu


def kernel(x, embedding_table):
    seq_len = x.shape[1]
    table_rows, d_model = embedding_table.shape

    rows_per_class = seq_len // 8

    def body(emb_ref, out_ref, scratch_ref):
        c = pl.program_id(0)
        a = pl.program_id(1)

        @pl.when(a == 0)
        def _build():
            for cs in range(8):
                @pl.when(c == cs)
                def _():
                    scratch_ref[...] = emb_ref[cs:cs + 2 * seq_len, :]

        off = seq_len - ((8 - c) % 8) - 8 * a - c
        off = pl.multiple_of(off, 8)
        out_ref[0] = scratch_ref[pl.ds(off, seq_len), :]

    return pl.pallas_call(
        body,
        grid=(8, rows_per_class),
        in_specs=[pl.BlockSpec((table_rows, d_model), lambda c, a: (0, 0))],
        out_specs=pl.BlockSpec(
            (1, seq_len, d_model),
            lambda c, a: (((8 - c) % 8) + 8 * a, 0, 0),
        ),
        out_shape=jax.ShapeDtypeStruct((seq_len, seq_len, d_model), jnp.float32),
        scratch_shapes=[pltpu.VMEM((2 * seq_len, d_model), jnp.float32)],
    )(embedding_table)


# speedup vs baseline: 7.1496x; 1.3498x over previous
import jax
import jax.numpy as jnp
from jax.experimental import pallas as pl
from jax.experimental.pallas import tpu as pltpu


def kernel(x, embedding_table):
    seq_len = x.shape[1]
    table_rows, d_model = embedding_table.shape

    rows_per_block = 4
    num_blocks = seq_len // rows_per_block

    def body(emb_ref, out_ref, scratch_ref):
        b = pl.program_id(0)

        @pl.when(b == 0)
        def _build():
            for cs in range(8):
                scratch_ref[cs] = emb_ref[cs:cs + 2 * seq_len, :]

        for r in range(rows_per_block):
            i = b * rows_per_block + r
            start = seq_len - i
            c = jax.lax.rem(start, 8)
            off = pl.multiple_of(start - c, 8)
            out_ref[r] = scratch_ref[c, pl.ds(off, seq_len), :]

    return pl.pallas_call(
        body,
        grid=(num_blocks,),
        in_specs=[pl.BlockSpec((table_rows, d_model), lambda b: (0, 0))],
        out_specs=pl.BlockSpec(
            (rows_per_block, seq_len, d_model), lambda b: (b, 0, 0)
        ),
        out_shape=jax.ShapeDtypeStruct((seq_len, seq_len, d_model), jnp.float32),
        scratch_shapes=[pltpu.VMEM((8, 2 * seq_len, d_model), jnp.float32)],
    )(embedding_table)
